# baseline (speedup 1.0000x reference)
import jax
import jax.numpy as jnp
from jax import lax
from jax.experimental import pallas as pl
from jax.experimental.pallas import tpu as pltpu
from jax.experimental.pallas import tpu_sc as plsc

BATCH = 1024
MAX_WORDS = 50
MAX_CHARS = 20
EMBED_DIM = 64
VOCAB = 1001

NC, NS, L = 2, 16, 16
NW = NC * NS
TOTAL_WORDS = BATCH * MAX_WORDS
WPT = TOTAL_WORDS // NW
CHUNK = 400
NCHUNK = WPT // CHUNK


def _sc_body(chars_hbm, table_hbm, out_hbm, table_v, chars_v, out_v):
    wid = lax.axis_index("s") * NC + lax.axis_index("c")
    pltpu.sync_copy(table_hbm, table_v)
    iota = lax.iota(jnp.int32, L)
    word0 = wid * WPT
    for k in range(NCHUNK):
        cbase = word0 + k * CHUNK
        pltpu.sync_copy(
            chars_hbm.at[pl.ds(cbase * MAX_CHARS, CHUNK * MAX_CHARS)], chars_v
        )

        DB = 32

        def group_body(g, _):
            gw = g * L
            cidx = iota * MAX_CHARS + gw * MAX_CHARS
            obase = iota * EMBED_DIM + gw * EMBED_DIM
            for half in range(EMBED_DIM // DB):
                dbase = half * DB
                row0 = plsc.load_gather(chars_v, [cidx]) * EMBED_DIM
                accs = tuple(
                    plsc.load_gather(table_v, [row0 + (dbase + j)])
                    for j in range(DB)
                )

                def c_body(c, accs):
                    row = plsc.load_gather(chars_v, [cidx + c]) * EMBED_DIM
                    return tuple(
                        jnp.maximum(
                            accs[j],
                            plsc.load_gather(table_v, [row + (dbase + j)]),
                        )
                        for j in range(DB)
                    )

                accs = lax.fori_loop(1, MAX_CHARS, c_body, accs)
                for j in range(DB):
                    plsc.store_scatter(out_v, [obase + (dbase + j)], accs[j])
            return 0

        lax.fori_loop(0, CHUNK // L, group_body, 0)
        pltpu.sync_copy(
            out_v, out_hbm.at[pl.ds(cbase * EMBED_DIM, CHUNK * EMBED_DIM)]
        )


def kernel(words, chars, table):
    del words
    chars_flat = chars.reshape(-1).astype(jnp.int32)
    table_flat = table.reshape(-1)
    mesh = plsc.VectorSubcoreMesh(core_axis_name="c", subcore_axis_name="s")
    run = pl.kernel(
        _sc_body,
        out_type=jax.ShapeDtypeStruct((TOTAL_WORDS * EMBED_DIM,), jnp.float32),
        mesh=mesh,
        scratch_types=[
            pltpu.VMEM((VOCAB * EMBED_DIM,), jnp.float32),
            pltpu.VMEM((CHUNK * MAX_CHARS,), jnp.int32),
            pltpu.VMEM((CHUNK * EMBED_DIM,), jnp.float32),
        ],
        compiler_params=pltpu.CompilerParams(needs_layout_passes=False),
    )
    out = run(chars_flat, table_flat)
    return out.reshape(BATCH, MAX_WORDS, EMBED_DIM)


# speedup vs baseline: 3.3162x; 3.3162x over previous
import jax
import jax.numpy as jnp
from jax import lax
from jax.experimental import pallas as pl
from jax.experimental.pallas import tpu as pltpu
from jax.experimental.pallas import tpu_sc as plsc

BATCH = 1024
MAX_WORDS = 50
MAX_CHARS = 20
EMBED_DIM = 64
VOCAB = 1001
STRIDE = EMBED_DIM + 1

NC, NS, L = 2, 16, 16
NW = NC * NS
TOTAL_WORDS = BATCH * MAX_WORDS
WPT = TOTAL_WORDS // NW
CHUNK = 400
NCHUNK = WPT // CHUNK
DB = 32


def _sc_body(chars_hbm, table_hbm, out_hbm, table_v, chars_v, out_v):
    wid = lax.axis_index("s") * NC + lax.axis_index("c")
    pltpu.sync_copy(table_hbm, table_v)
    iota = lax.iota(jnp.int32, L)
    word0 = wid * WPT
    for k in range(NCHUNK):
        cbase = word0 + k * CHUNK
        pltpu.sync_copy(
            chars_hbm.at[pl.ds(cbase * MAX_CHARS, CHUNK * MAX_CHARS)], chars_v
        )

        def group_body(g, _):
            gw = g * L
            cidx = iota * MAX_CHARS + gw * MAX_CHARS
            obase = (iota + gw) * STRIDE
            for half in range(EMBED_DIM // DB):
                dbase = half * DB
                row0 = plsc.load_gather(chars_v, [cidx]) * STRIDE
                accs = tuple(
                    plsc.load_gather(table_v, [row0 + (dbase + j)])
                    for j in range(DB)
                )

                def c_body(c, accs):
                    row = plsc.load_gather(chars_v, [cidx + c]) * STRIDE
                    return tuple(
                        jnp.maximum(
                            accs[j],
                            plsc.load_gather(table_v, [row + (dbase + j)]),
                        )
                        for j in range(DB)
                    )

                accs = lax.fori_loop(1, MAX_CHARS, c_body, accs)
                for j in range(DB):
                    plsc.store_scatter(out_v, [obase + (dbase + j)], accs[j])
            return 0

        lax.fori_loop(0, CHUNK // L, group_body, 0)
        pltpu.sync_copy(
            out_v, out_hbm.at[pl.ds(cbase * STRIDE, CHUNK * STRIDE)]
        )


def kernel(words, chars, table):
    del words
    chars_flat = chars.reshape(-1).astype(jnp.int32)
    table_pad = jnp.pad(table, ((0, 0), (0, STRIDE - EMBED_DIM))).reshape(-1)
    mesh = plsc.VectorSubcoreMesh(core_axis_name="c", subcore_axis_name="s")
    run = pl.kernel(
        _sc_body,
        out_type=jax.ShapeDtypeStruct((TOTAL_WORDS * STRIDE,), jnp.float32),
        mesh=mesh,
        scratch_types=[
            pltpu.VMEM((VOCAB * STRIDE,), jnp.float32),
            pltpu.VMEM((CHUNK * MAX_CHARS,), jnp.int32),
            pltpu.VMEM((CHUNK * STRIDE,), jnp.float32),
        ],
        compiler_params=pltpu.CompilerParams(needs_layout_passes=False),
    )
    out = run(chars_flat, table_pad)
    out = out.reshape(TOTAL_WORDS, STRIDE)[:, :EMBED_DIM]
    return out.reshape(BATCH, MAX_WORDS, EMBED_DIM)


# speedup vs baseline: 3.3701x; 1.0162x over previous
import jax
import jax.numpy as jnp
from jax import lax
from jax.experimental import pallas as pl
from jax.experimental.pallas import tpu as pltpu
from jax.experimental.pallas import tpu_sc as plsc

BATCH = 1024
MAX_WORDS = 50
MAX_CHARS = 20
EMBED_DIM = 64
VOCAB = 1001
PK = EMBED_DIM // 2
STRIDE = PK + 1

NC, NS, L = 2, 16, 16
NW = NC * NS
TOTAL_WORDS = BATCH * MAX_WORDS
WPT = TOTAL_WORDS // NW
CHUNK = 400
NCHUNK = WPT // CHUNK


def _sc_body(chars_hbm, table_hbm, out_hbm, table_v, chars_v, out_v):
    wid = lax.axis_index("s") * NC + lax.axis_index("c")
    pltpu.sync_copy(table_hbm, table_v)
    iota = lax.iota(jnp.int32, L)
    word0 = wid * WPT
    for k in range(NCHUNK):
        cbase = word0 + k * CHUNK
        pltpu.sync_copy(
            chars_hbm.at[pl.ds(cbase * MAX_CHARS, CHUNK * MAX_CHARS)], chars_v
        )

        def group_body(g, _):
            gw = g * L
            cidx = iota * MAX_CHARS + gw * MAX_CHARS
            obase = (iota + gw) * STRIDE
            row0 = plsc.load_gather(chars_v, [cidx]) * STRIDE
            accs = tuple(
                plsc.bitcast(
                    plsc.load_gather(table_v, [row0 + j]), jnp.bfloat16
                )
                for j in range(PK)
            )

            def c_body(c, accs):
                row = plsc.load_gather(chars_v, [cidx + c]) * STRIDE
                return tuple(
                    jnp.maximum(
                        accs[j],
                        plsc.bitcast(
                            plsc.load_gather(table_v, [row + j]), jnp.bfloat16
                        ),
                    )
                    for j in range(PK)
                )

            accs = lax.fori_loop(1, MAX_CHARS, c_body, accs)
            for j in range(PK):
                plsc.store_scatter(
                    out_v, [obase + j], plsc.bitcast(accs[j], jnp.int32)
                )
            return 0

        lax.fori_loop(0, CHUNK // L, group_body, 0)
        pltpu.sync_copy(
            out_v, out_hbm.at[pl.ds(cbase * STRIDE, CHUNK * STRIDE)]
        )


def kernel(words, chars, table):
    del words
    chars_flat = chars.reshape(-1).astype(jnp.int32)
    table_bf = table.astype(jnp.bfloat16).reshape(VOCAB, PK, 2)
    table_pk = lax.bitcast_convert_type(table_bf, jnp.int32)
    table_pk = jnp.pad(table_pk, ((0, 0), (0, STRIDE - PK))).reshape(-1)
    mesh = plsc.VectorSubcoreMesh(core_axis_name="c", subcore_axis_name="s")
    run = pl.kernel(
        _sc_body,
        out_type=jax.ShapeDtypeStruct((TOTAL_WORDS * STRIDE,), jnp.int32),
        mesh=mesh,
        scratch_types=[
            pltpu.VMEM((VOCAB * STRIDE,), jnp.int32),
            pltpu.VMEM((CHUNK * MAX_CHARS,), jnp.int32),
            pltpu.VMEM((CHUNK * STRIDE,), jnp.int32),
        ],
        compiler_params=pltpu.CompilerParams(needs_layout_passes=False),
    )
    out = run(chars_flat, table_pk)
    out = out.reshape(TOTAL_WORDS, STRIDE)[:, :PK]
    out = lax.bitcast_convert_type(out, jnp.bfloat16).astype(jnp.float32)
    return out.reshape(BATCH, MAX_WORDS, EMBED_DIM)


# speedup vs baseline: 3.6658x; 1.0877x over previous
import jax
import jax.numpy as jnp
from jax import lax
from jax.experimental import pallas as pl
from jax.experimental.pallas import tpu as pltpu
from jax.experimental.pallas import tpu_sc as plsc

BATCH = 1024
MAX_WORDS = 50
MAX_CHARS = 20
EMBED_DIM = 64
VOCAB = 1001
PK = EMBED_DIM // 2
TSTRIDE = PK + 1

NC, NS, L = 2, 16, 16
NW = NC * NS
TOTAL_WORDS = BATCH * MAX_WORDS
WPT = TOTAL_WORDS // NW
CHUNK = 400
DSTRIDE = CHUNK + 1
NCHUNK = WPT // CHUNK


def _sc_body(chars_hbm, table_hbm, out_hbm, table_v, chars_v, dm_v, out_v):
    wid = lax.axis_index("s") * NC + lax.axis_index("c")
    pltpu.sync_copy(table_hbm, table_v)
    iota = lax.iota(jnp.int32, L)
    word0 = wid * WPT
    for k in range(NCHUNK):
        cbase = word0 + k * CHUNK
        pltpu.sync_copy(
            chars_hbm.at[pl.ds(cbase * MAX_CHARS, CHUNK * MAX_CHARS)], chars_v
        )

        def group_body(g, _):
            gw = g * L
            cidx = iota * MAX_CHARS + gw * MAX_CHARS
            row0 = plsc.load_gather(chars_v, [cidx]) * TSTRIDE
            accs = tuple(
                plsc.bitcast(
                    plsc.load_gather(table_v, [row0 + j]), jnp.bfloat16
                )
                for j in range(PK)
            )

            def c_body(c, accs):
                row = plsc.load_gather(chars_v, [cidx + c]) * TSTRIDE
                return tuple(
                    jnp.maximum(
                        accs[j],
                        plsc.bitcast(
                            plsc.load_gather(table_v, [row + j]), jnp.bfloat16
                        ),
                    )
                    for j in range(PK)
                )

            accs = lax.fori_loop(1, MAX_CHARS, c_body, accs)
            for j in range(PK):
                dm_v[pl.ds(j * DSTRIDE + gw, L)] = plsc.bitcast(
                    accs[j], jnp.int32
                )
            return 0

        lax.fori_loop(0, CHUNK // L, group_body, 0)

        def comp_body(w, _):
            for j2 in range(PK // L):
                v = plsc.load_gather(
                    dm_v, [(j2 * L + iota) * DSTRIDE + w]
                )
                out_v[pl.ds(w * PK + j2 * L, L)] = v
            return 0

        lax.fori_loop(0, CHUNK, comp_body, 0, unroll=4)
        pltpu.sync_copy(out_v, out_hbm.at[pl.ds(cbase * PK, CHUNK * PK)])


def kernel(words, chars, table):
    del words
    chars_flat = chars.reshape(-1).astype(jnp.int32)
    table_bf = table.astype(jnp.bfloat16).reshape(VOCAB, PK, 2)
    table_pk = lax.bitcast_convert_type(table_bf, jnp.int32)
    table_pk = jnp.pad(table_pk, ((0, 0), (0, TSTRIDE - PK))).reshape(-1)
    mesh = plsc.VectorSubcoreMesh(core_axis_name="c", subcore_axis_name="s")
    run = pl.kernel(
        _sc_body,
        out_type=jax.ShapeDtypeStruct((TOTAL_WORDS * PK,), jnp.int32),
        mesh=mesh,
        scratch_types=[
            pltpu.VMEM((VOCAB * TSTRIDE,), jnp.int32),
            pltpu.VMEM((CHUNK * MAX_CHARS,), jnp.int32),
            pltpu.VMEM((PK * DSTRIDE,), jnp.int32),
            pltpu.VMEM((CHUNK * PK,), jnp.int32),
        ],
        compiler_params=pltpu.CompilerParams(needs_layout_passes=False),
    )
    out = run(chars_flat, table_pk)
    out = lax.bitcast_convert_type(
        out.reshape(TOTAL_WORDS, PK), jnp.bfloat16
    ).astype(jnp.float32)
    return out.reshape(BATCH, MAX_WORDS, EMBED_DIM)


# speedup vs baseline: 4.6246x; 1.2616x over previous
import jax
import jax.numpy as jnp
from jax import lax
from jax.experimental import pallas as pl
from jax.experimental.pallas import tpu as pltpu
from jax.experimental.pallas import tpu_sc as plsc

BATCH = 1024
MAX_WORDS = 50
MAX_CHARS = 20
EMBED_DIM = 64
VOCAB = 1001
PK = EMBED_DIM // 2
TSTRIDE = PK + 1

NC, NS, L = 2, 16, 16
NW = NC * NS
TOTAL_WORDS = BATCH * MAX_WORDS
WPT = TOTAL_WORDS // NW
CHUNK = 400
DSTRIDE = CHUNK + 1
NCHUNK = WPT // CHUNK


def _sc_body(chars_hbm, table_hbm, out_hbm, table_v, chars_v, dm_v, out_v):
    wid = lax.axis_index("s") * NC + lax.axis_index("c")
    pltpu.sync_copy(table_hbm, table_v)
    iota = lax.iota(jnp.int32, L)
    word0 = wid * WPT
    for k in range(NCHUNK):
        cbase = word0 + k * CHUNK
        pltpu.sync_copy(
            chars_hbm.at[pl.ds(cbase * MAX_CHARS, CHUNK * MAX_CHARS)], chars_v
        )

        def group_body(g, _):
            gw = g * L
            cidx = iota * MAX_CHARS + gw * MAX_CHARS
            row0 = plsc.load_gather(chars_v, [cidx]) * TSTRIDE
            accs = tuple(
                plsc.bitcast(
                    plsc.load_gather(table_v, [row0 + j]), jnp.bfloat16
                )
                for j in range(PK)
            )

            def c_body(c, accs):
                row = plsc.load_gather(chars_v, [cidx + c]) * TSTRIDE
                return tuple(
                    jnp.maximum(
                        accs[j],
                        plsc.bitcast(
                            plsc.load_gather(table_v, [row + j]), jnp.bfloat16
                        ),
                    )
                    for j in range(PK)
                )

            accs = lax.fori_loop(1, MAX_CHARS, c_body, accs)
            for j in range(PK):
                lo, hi = plsc.unpack(accs[j], format=plsc.PackFormat.INTERLEAVED)
                dm_v[pl.ds((2 * j) * DSTRIDE + gw, L)] = lo
                dm_v[pl.ds((2 * j + 1) * DSTRIDE + gw, L)] = hi
            return 0

        lax.fori_loop(0, CHUNK // L, group_body, 0)

        def comp_body(w, _):
            for j2 in range(EMBED_DIM // L):
                v = plsc.load_gather(dm_v, [(j2 * L + iota) * DSTRIDE + w])
                out_v[pl.ds(w * EMBED_DIM + j2 * L, L)] = v
            return 0

        lax.fori_loop(0, CHUNK, comp_body, 0, unroll=4)
        pltpu.sync_copy(
            out_v, out_hbm.at[pl.ds(cbase * EMBED_DIM, CHUNK * EMBED_DIM)]
        )


def kernel(words, chars, table):
    del words
    chars_flat = chars.reshape(-1).astype(jnp.int32)
    table_bf = table.astype(jnp.bfloat16).reshape(VOCAB, PK, 2)
    table_pk = lax.bitcast_convert_type(table_bf, jnp.int32)
    table_pk = jnp.pad(table_pk, ((0, 0), (0, TSTRIDE - PK))).reshape(-1)
    mesh = plsc.VectorSubcoreMesh(core_axis_name="c", subcore_axis_name="s")
    run = pl.kernel(
        _sc_body,
        out_type=jax.ShapeDtypeStruct((TOTAL_WORDS * EMBED_DIM,), jnp.float32),
        mesh=mesh,
        scratch_types=[
            pltpu.VMEM((VOCAB * TSTRIDE,), jnp.int32),
            pltpu.VMEM((CHUNK * MAX_CHARS,), jnp.int32),
            pltpu.VMEM((EMBED_DIM * DSTRIDE,), jnp.float32),
            pltpu.VMEM((CHUNK * EMBED_DIM,), jnp.float32),
        ],
        compiler_params=pltpu.CompilerParams(needs_layout_passes=False),
    )
    out = run(chars_flat, table_pk)
    return out.reshape(BATCH, MAX_WORDS, EMBED_DIM)
